# baseline (speedup 1.0000x reference)
import jax
import jax.numpy as jnp
from jax.experimental import pallas as pl
from jax.experimental.pallas import tpu as pltpu

NMS_THRESH = 0.3
N = 1000
NP = 1024
C = 151
CH = 32
NW = NP // CH
JT = 256
NJ = NP // JT


def _nms_kernel(p_ref, keep_ref, m_ref, pk_ref):
    c = pl.program_id(0)

    @pl.when(c == 0)
    def _init():
        for jt in range(NJ):
            g_hi = 8 * jt + 8
            if g_hi < NW:
                m_ref[g_hi:NW, jt * JT:(jt + 1) * JT] = jnp.zeros(
                    (NW - g_hi, JT), jnp.int32)
        ii = jax.lax.broadcasted_iota(jnp.int32, (NP, 2 * NW), 0)
        gg = jax.lax.broadcasted_iota(jnp.int32, (NP, 2 * NW), 1)
        word = jax.lax.shift_right_logical(ii, 5)
        sel = (word == (gg & (NW - 1))) & ((gg >= NW) == ((ii & 16) != 0))
        pw = jax.lax.shift_left(jnp.int32(1), ii & 15)
        pk_ref[...] = jnp.where(sel, pw, 0).astype(jnp.float32)

    p = p_ref[0]
    x1r = p[0:1, :]
    y1r = p[1:2, :]
    x2r = p[2:3, :]
    y2r = p[3:4, :]
    ar = (x2r - x1r + 1.0) * (y2r - y1r + 1.0)

    q = jnp.concatenate([p[0:4, :], ar, jnp.zeros((3, NP), jnp.float32)],
                        axis=0)
    qT = q.T

    rr = jax.lax.broadcasted_iota(jnp.int32, (2, CH), 1)
    hh = jax.lax.broadcasted_iota(jnp.int32, (2, CH), 0)
    p2 = jnp.where((rr >= 16) == (hh == 1),
                   jax.lax.shift_left(jnp.int32(1), rr & 15),
                   0).astype(jnp.float32)

    for jt in range(NJ):
        j0 = jt * JT
        sl = slice(j0, j0 + JT)
        x1rt = x1r[:, sl]
        y1rt = y1r[:, sl]
        x2rt = x2r[:, sl]
        y2rt = y2r[:, sl]
        art = ar[:, sl]
        for g in range(8 * jt + 8):
            r0 = g * CH
            qc = qT[r0:r0 + CH]
            xx1 = jnp.maximum(qc[:, 0:1], x1rt)
            yy1 = jnp.maximum(qc[:, 1:2], y1rt)
            xx2 = jnp.minimum(qc[:, 2:3], x2rt)
            yy2 = jnp.minimum(qc[:, 3:4], y2rt)
            w = jnp.maximum(0.0, xx2 - xx1 + 1.0)
            h = yy2 - yy1 + 1.0
            inter = w * h
            iou = inter / (qc[:, 4:5] + art - inter)
            iouf = iou > NMS_THRESH
            if r0 + CH - 1 < j0:
                mb = iouf.astype(jnp.float32)
            else:
                ri = jax.lax.broadcasted_iota(jnp.int32, (CH, JT), 0) + r0
                cj = jax.lax.broadcasted_iota(jnp.int32, (CH, JT), 1) + j0
                mb = (iouf & (ri < cj)).astype(jnp.float32)
            lohi = jnp.dot(p2, mb,
                           preferred_element_type=jnp.float32)
            word = (lohi[0:1].astype(jnp.int32)
                    | jax.lax.shift_left(lohi[1:2].astype(jnp.int32), 16))
            m_ref[g:g + 1, sl] = word

    def bstep(kb):
        anded = m_ref[...] & kb
        t = anded[0:16] | anded[16:32]
        t = t[0:8] | t[8:16]
        t = t[0:4] | t[4:8]
        t = t[0:2] | t[2:4]
        orr = t[0:1] | t[1:2]
        kf = (orr == 0).astype(jnp.float32)
        packed = jnp.dot(kf, pk_ref[...],
                         preferred_element_type=jnp.float32)
        lo = packed[:, 0:NW].astype(jnp.int32)
        hi = packed[:, NW:2 * NW].astype(jnp.int32)
        kb_new = (lo | jax.lax.shift_left(hi, 16)).T
        return kb_new, kf

    kb = jnp.full((NW, 1), -1, jnp.int32)
    kf = jnp.ones((1, NP), jnp.float32)
    for _ in range(5):
        kb, kf = bstep(kb)

    def wbody(carry):
        kb0, _, _ = carry
        kb1, _ = bstep(kb0)
        kb2, kf2 = bstep(kb1)
        return kb2, kf2, jnp.any(kb2 != kb0)

    kb, kf, _ = jax.lax.while_loop(lambda carry: carry[2], wbody,
                                   (kb, kf, jnp.bool_(True)))

    keep_ref[0] = kf


def _relhead_kernel(vr_ref, w_ref, b_ref, out_ref):
    acc = jax.lax.dot_general(
        vr_ref[...], w_ref[...],
        dimension_numbers=(((1,), (1,)), ((), ())),
        preferred_element_type=jnp.float32,
    )
    out_ref[...] = acc + b_ref[...]


@jax.jit
def kernel(obj_logits, vr, boxes_per_cls, W, b):
    probs = jax.nn.softmax(obj_logits, axis=1)
    pf = probs[:, 1:]
    order = jnp.argsort(-pf, axis=0)
    bs = jnp.take_along_axis(boxes_per_cls[:, 1:, :], order[:, :, None],
                             axis=0)

    bT = jnp.transpose(bs, (1, 2, 0))
    bT = jnp.pad(bT, ((0, 0), (0, 0), (0, NP - N)))
    pad = jnp.zeros((C - 1, 4, NP), jnp.float32)
    packed = jnp.concatenate([bT, pad], axis=1)

    keep = pl.pallas_call(
        _nms_kernel,
        grid=(C - 1,),
        in_specs=[pl.BlockSpec((1, 8, NP), lambda c: (c, 0, 0))],
        out_specs=pl.BlockSpec((1, 1, NP), lambda c: (c, 0, 0)),
        out_shape=jax.ShapeDtypeStruct((C - 1, 1, NP), jnp.float32),
        scratch_shapes=[pltpu.VMEM((NW, NP), jnp.int32),
                        pltpu.VMEM((NP, 2 * NW), jnp.float32)],
    )(packed)

    keep_s = keep[:, 0, :N]
    colidx = jnp.broadcast_to(jnp.arange(C - 1)[None, :], (N, C - 1))
    nms_f = jnp.zeros((N, C - 1), jnp.float32).at[order, colidx].set(keep_s.T)
    obj_preds = jnp.argmax(nms_f * pf, axis=1) + 1

    RB = 400
    rel_dists = pl.pallas_call(
        _relhead_kernel,
        grid=(vr.shape[0] // RB,),
        in_specs=[pl.BlockSpec((RB, vr.shape[1]), lambda i: (i, 0)),
                  pl.BlockSpec(W.shape, lambda i: (0, 0)),
                  pl.BlockSpec((1, W.shape[0]), lambda i: (0, 0))],
        out_specs=pl.BlockSpec((RB, W.shape[0]), lambda i: (i, 0)),
        out_shape=jax.ShapeDtypeStruct((vr.shape[0], W.shape[0]), jnp.float32),
    )(vr, W, b.reshape(1, -1))

    return (obj_logits, obj_preds, rel_dists)


# speedup vs baseline: 1.9136x; 1.9136x over previous
import jax
import jax.numpy as jnp
from jax.experimental import pallas as pl
from jax.experimental.pallas import tpu as pltpu

NMS_THRESH = 0.3
N = 1000
NP = 1024
C = 151
CH = 32
NW = NP // CH
JT = 256
NJ = NP // JT


def _nms_kernel(p_ref, keep_ref, m_ref, pk_ref):
    c = pl.program_id(0)

    @pl.when(c == 0)
    def _init():
        for jt in range(NJ):
            g_hi = 8 * jt + 8
            if g_hi < NW:
                m_ref[g_hi:NW, jt * JT:(jt + 1) * JT] = jnp.zeros(
                    (NW - g_hi, JT), jnp.int32)
        ii = jax.lax.broadcasted_iota(jnp.int32, (NP, 2 * NW), 0)
        gg = jax.lax.broadcasted_iota(jnp.int32, (NP, 2 * NW), 1)
        word = jax.lax.shift_right_logical(ii, 5)
        sel = (word == (gg & (NW - 1))) & ((gg >= NW) == ((ii & 16) != 0))
        pw = jax.lax.shift_left(jnp.int32(1), ii & 15)
        pk_ref[...] = jnp.where(sel, pw, 0).astype(jnp.float32)

    p = p_ref[0]
    x1r = p[0:1, :]
    y1r = p[1:2, :]
    x2r = p[2:3, :]
    y2r = p[3:4, :]
    ar = (x2r - x1r + 1.0) * (y2r - y1r + 1.0)

    q = jnp.concatenate([p[0:4, :], ar, jnp.zeros((3, NP), jnp.float32)],
                        axis=0)
    qT = q.T

    rr = jax.lax.broadcasted_iota(jnp.int32, (2, CH), 1)
    hh = jax.lax.broadcasted_iota(jnp.int32, (2, CH), 0)
    p2 = jnp.where((rr >= 16) == (hh == 1),
                   jax.lax.shift_left(jnp.int32(1), rr & 15),
                   0).astype(jnp.float32)

    for jt in range(NJ):
        j0 = jt * JT
        sl = slice(j0, j0 + JT)
        x1rt = x1r[:, sl]
        y1rt = y1r[:, sl]
        x2rt = x2r[:, sl]
        y2rt = y2r[:, sl]
        art = ar[:, sl]
        for g in range(8 * jt + 8):
            r0 = g * CH
            qc = qT[r0:r0 + CH]
            xx1 = jnp.maximum(qc[:, 0:1], x1rt)
            yy1 = jnp.maximum(qc[:, 1:2], y1rt)
            xx2 = jnp.minimum(qc[:, 2:3], x2rt)
            yy2 = jnp.minimum(qc[:, 3:4], y2rt)
            w = jnp.maximum(0.0, xx2 - xx1 + 1.0)
            h = yy2 - yy1 + 1.0
            inter = w * h
            iou = inter / (qc[:, 4:5] + art - inter)
            iouf = iou > NMS_THRESH
            if r0 + CH - 1 < j0:
                mb = iouf.astype(jnp.float32)
            else:
                ri = jax.lax.broadcasted_iota(jnp.int32, (CH, JT), 0) + r0
                cj = jax.lax.broadcasted_iota(jnp.int32, (CH, JT), 1) + j0
                mb = (iouf & (ri < cj)).astype(jnp.float32)
            lohi = jnp.dot(p2, mb,
                           preferred_element_type=jnp.float32)
            word = (lohi[0:1].astype(jnp.int32)
                    | jax.lax.shift_left(lohi[1:2].astype(jnp.int32), 16))
            m_ref[g:g + 1, sl] = word

    def bstep(kb):
        anded = m_ref[...] & kb
        t = anded[0:16] | anded[16:32]
        t = t[0:8] | t[8:16]
        t = t[0:4] | t[4:8]
        t = t[0:2] | t[2:4]
        orr = t[0:1] | t[1:2]
        kf = (orr == 0).astype(jnp.float32)
        packed = jnp.dot(kf, pk_ref[...],
                         preferred_element_type=jnp.float32)
        lo = packed[:, 0:NW].astype(jnp.int32)
        hi = packed[:, NW:2 * NW].astype(jnp.int32)
        kb_new = (lo | jax.lax.shift_left(hi, 16)).T
        return kb_new, kf

    kb = jnp.full((NW, 1), -1, jnp.int32)
    kf = jnp.ones((1, NP), jnp.float32)
    for _ in range(5):
        kb, kf = bstep(kb)

    def wbody(carry):
        kb0, _, _ = carry
        kb1, _ = bstep(kb0)
        kb2, kf2 = bstep(kb1)
        return kb2, kf2, jnp.any(kb2 != kb0)

    kb, kf, _ = jax.lax.while_loop(lambda carry: carry[2], wbody,
                                   (kb, kf, jnp.bool_(True)))

    keep_ref[0] = kf


def _relhead_kernel(vr_ref, w_ref, b_ref, out_ref):
    acc = jax.lax.dot_general(
        vr_ref[...], w_ref[...],
        dimension_numbers=(((1,), (1,)), ((), ())),
        preferred_element_type=jnp.float32,
    )
    out_ref[...] = acc + b_ref[...]


@jax.jit
def kernel(obj_logits, vr, boxes_per_cls, W, b):
    probs = jax.nn.softmax(obj_logits, axis=1)
    pf = probs[:, 1:]
    key = -pf.T
    idxT = jnp.broadcast_to(jnp.arange(N, dtype=jnp.int32)[None, :],
                            (C - 1, N))
    coordsT = [boxes_per_cls[:, 1:, k].T for k in range(4)]
    _, sx1, sy1, sx2, sy2, sidx = jax.lax.sort(
        [key] + coordsT + [idxT], dimension=1, is_stable=True, num_keys=1)

    bT = jnp.stack([sx1, sy1, sx2, sy2], axis=1)
    bT = jnp.pad(bT, ((0, 0), (0, 0), (0, NP - N)))
    pad = jnp.zeros((C - 1, 4, NP), jnp.float32)
    packed = jnp.concatenate([bT, pad], axis=1)

    keep = pl.pallas_call(
        _nms_kernel,
        grid=(C - 1,),
        in_specs=[pl.BlockSpec((1, 8, NP), lambda c: (c, 0, 0))],
        out_specs=pl.BlockSpec((1, 1, NP), lambda c: (c, 0, 0)),
        out_shape=jax.ShapeDtypeStruct((C - 1, 1, NP), jnp.float32),
        scratch_shapes=[pltpu.VMEM((NW, NP), jnp.int32),
                        pltpu.VMEM((NP, 2 * NW), jnp.float32)],
    )(packed)

    keep_s = keep[:, 0, :N]
    _, keep_orig = jax.lax.sort([sidx, keep_s], dimension=1, num_keys=1)
    obj_preds = jnp.argmax(keep_orig.T * pf, axis=1) + 1

    RB = 400
    rel_dists = pl.pallas_call(
        _relhead_kernel,
        grid=(vr.shape[0] // RB,),
        in_specs=[pl.BlockSpec((RB, vr.shape[1]), lambda i: (i, 0)),
                  pl.BlockSpec(W.shape, lambda i: (0, 0)),
                  pl.BlockSpec((1, W.shape[0]), lambda i: (0, 0))],
        out_specs=pl.BlockSpec((RB, W.shape[0]), lambda i: (i, 0)),
        out_shape=jax.ShapeDtypeStruct((vr.shape[0], W.shape[0]), jnp.float32),
    )(vr, W, b.reshape(1, -1))

    return (obj_logits, obj_preds, rel_dists)


# speedup vs baseline: 1.9158x; 1.0011x over previous
import jax
import jax.numpy as jnp
from jax.experimental import pallas as pl
from jax.experimental.pallas import tpu as pltpu

NMS_THRESH = 0.3
N = 1000
NP = 1024
C = 151
CH = 32
NW = NP // CH
JT = 256
NJ = NP // JT


def _nms_kernel(x1_ref, y1_ref, x2_ref, y2_ref, keep_ref, m_ref, pk_ref):
    c = pl.program_id(0)

    @pl.when(c == 0)
    def _init():
        for jt in range(NJ):
            g_hi = 8 * jt + 8
            if g_hi < NW:
                m_ref[g_hi:NW, jt * JT:(jt + 1) * JT] = jnp.zeros(
                    (NW - g_hi, JT), jnp.int32)
        ii = jax.lax.broadcasted_iota(jnp.int32, (NP, 2 * NW), 0)
        gg = jax.lax.broadcasted_iota(jnp.int32, (NP, 2 * NW), 1)
        word = jax.lax.shift_right_logical(ii, 5)
        sel = (word == (gg & (NW - 1))) & ((gg >= NW) == ((ii & 16) != 0))
        pw = jax.lax.shift_left(jnp.int32(1), ii & 15)
        pk_ref[...] = jnp.where(sel, pw, 0).astype(jnp.float32)

    x1r = x1_ref[0]
    y1r = y1_ref[0]
    x2r = x2_ref[0]
    y2r = y2_ref[0]
    ar = (x2r - x1r + 1.0) * (y2r - y1r + 1.0)

    q = jnp.concatenate([x1r, y1r, x2r, y2r, ar,
                         jnp.zeros((3, NP), jnp.float32)], axis=0)
    qT = q.T

    rr = jax.lax.broadcasted_iota(jnp.int32, (2, CH), 1)
    hh = jax.lax.broadcasted_iota(jnp.int32, (2, CH), 0)
    p2 = jnp.where((rr >= 16) == (hh == 1),
                   jax.lax.shift_left(jnp.int32(1), rr & 15),
                   0).astype(jnp.float32)

    for jt in range(NJ):
        j0 = jt * JT
        sl = slice(j0, j0 + JT)
        x1rt = x1r[:, sl]
        y1rt = y1r[:, sl]
        x2rt = x2r[:, sl]
        y2rt = y2r[:, sl]
        art = ar[:, sl]
        for g in range(8 * jt + 8):
            r0 = g * CH
            qc = qT[r0:r0 + CH]
            xx1 = jnp.maximum(qc[:, 0:1], x1rt)
            yy1 = jnp.maximum(qc[:, 1:2], y1rt)
            xx2 = jnp.minimum(qc[:, 2:3], x2rt)
            yy2 = jnp.minimum(qc[:, 3:4], y2rt)
            w = jnp.maximum(0.0, xx2 - xx1 + 1.0)
            h = yy2 - yy1 + 1.0
            inter = w * h
            iou = inter / (qc[:, 4:5] + art - inter)
            iouf = iou > NMS_THRESH
            if r0 + CH - 1 < j0:
                mb = iouf.astype(jnp.float32)
            else:
                ri = jax.lax.broadcasted_iota(jnp.int32, (CH, JT), 0) + r0
                cj = jax.lax.broadcasted_iota(jnp.int32, (CH, JT), 1) + j0
                mb = (iouf & (ri < cj)).astype(jnp.float32)
            lohi = jnp.dot(p2, mb,
                           preferred_element_type=jnp.float32)
            word = (lohi[0:1].astype(jnp.int32)
                    | jax.lax.shift_left(lohi[1:2].astype(jnp.int32), 16))
            m_ref[g:g + 1, sl] = word

    def bstep(kb):
        anded = m_ref[...] & kb
        t = anded[0:16] | anded[16:32]
        t = t[0:8] | t[8:16]
        t = t[0:4] | t[4:8]
        t = t[0:2] | t[2:4]
        orr = t[0:1] | t[1:2]
        kf = (orr == 0).astype(jnp.float32)
        packed = jnp.dot(kf, pk_ref[...],
                         preferred_element_type=jnp.float32)
        lo = packed[:, 0:NW].astype(jnp.int32)
        hi = packed[:, NW:2 * NW].astype(jnp.int32)
        kb_new = (lo | jax.lax.shift_left(hi, 16)).T
        return kb_new, kf

    kb = jnp.full((NW, 1), -1, jnp.int32)
    kf = jnp.ones((1, NP), jnp.float32)
    for _ in range(5):
        kb, kf = bstep(kb)

    def wbody(carry):
        kb0, _, _ = carry
        kb1, _ = bstep(kb0)
        kb2, kf2 = bstep(kb1)
        return kb2, kf2, jnp.any(kb2 != kb0)

    kb, kf, _ = jax.lax.while_loop(lambda carry: carry[2], wbody,
                                   (kb, kf, jnp.bool_(True)))

    keep_ref[0] = kf


def _relhead_kernel(vr_ref, w_ref, b_ref, out_ref):
    acc = jax.lax.dot_general(
        vr_ref[...], w_ref[...],
        dimension_numbers=(((1,), (1,)), ((), ())),
        preferred_element_type=jnp.float32,
    )
    out_ref[...] = acc + b_ref[...]


@jax.jit
def kernel(obj_logits, vr, boxes_per_cls, W, b):
    probs = jax.nn.softmax(obj_logits, axis=1)
    pf = probs[:, 1:]
    key = -pf.T
    idxT = jnp.broadcast_to(jnp.arange(N, dtype=jnp.int32)[None, :],
                            (C - 1, N))
    coordsT = [boxes_per_cls[:, 1:, k].T for k in range(4)]
    _, sx1, sy1, sx2, sy2, sidx = jax.lax.sort(
        [key] + coordsT + [idxT], dimension=1, is_stable=True, num_keys=1)

    planes = [jnp.pad(a, ((0, 0), (0, NP - N))).reshape(C - 1, 1, NP)
              for a in (sx1, sy1, sx2, sy2)]

    cspec = pl.BlockSpec((1, 1, NP), lambda c: (c, 0, 0))
    keep = pl.pallas_call(
        _nms_kernel,
        grid=(C - 1,),
        in_specs=[cspec] * 4,
        out_specs=cspec,
        out_shape=jax.ShapeDtypeStruct((C - 1, 1, NP), jnp.float32),
        scratch_shapes=[pltpu.VMEM((NW, NP), jnp.int32),
                        pltpu.VMEM((NP, 2 * NW), jnp.float32)],
    )(*planes)

    keep_s = keep[:, 0, :N]
    _, keep_orig = jax.lax.sort([sidx, keep_s], dimension=1, num_keys=1)
    obj_preds = jnp.argmax(keep_orig.T * pf, axis=1) + 1

    RB = 400
    rel_dists = pl.pallas_call(
        _relhead_kernel,
        grid=(vr.shape[0] // RB,),
        in_specs=[pl.BlockSpec((RB, vr.shape[1]), lambda i: (i, 0)),
                  pl.BlockSpec(W.shape, lambda i: (0, 0)),
                  pl.BlockSpec((1, W.shape[0]), lambda i: (0, 0))],
        out_specs=pl.BlockSpec((RB, W.shape[0]), lambda i: (i, 0)),
        out_shape=jax.ShapeDtypeStruct((vr.shape[0], W.shape[0]), jnp.float32),
    )(vr, W, b.reshape(1, -1))

    return (obj_logits, obj_preds, rel_dists)
